# baseline (speedup 1.0000x reference)
import functools

import jax
import jax.numpy as jnp
from jax import lax
from jax.experimental import pallas as pl
from jax.experimental.pallas import tpu as pltpu
from jax.experimental.pallas import tpu_sc as plsc

_DIM = 128
_N = 10000
_NPAD = 10240
_DUMMY = 10000
_NC, _NS = 2, 16
_NW = _NC * _NS
_CH = 128


def _pick_block(n):
    for b in (512, 400, 320, 256, 200, 160, 128, 80, 64, 40, 32, 16, 8):
        if n % b == 0:
            return b
    raise ValueError(f"no row block for {n}")


def _stats_body(x_ref, o_ref):
    xb = x_ref[...]
    s = jnp.sum(xb, axis=0, keepdims=True)
    q = jnp.sum(xb * xb, axis=0, keepdims=True)

    @pl.when(pl.program_id(0) == 0)
    def _():
        o_ref[...] = jnp.zeros_like(o_ref)

    o_ref[...] += jnp.concatenate([s, q], axis=0)


def _col_stats(x):
    n, d = x.shape
    b = _pick_block(n)
    out = pl.pallas_call(
        _stats_body,
        grid=(n // b,),
        in_specs=[pl.BlockSpec((b, d), lambda i: (i, 0))],
        out_specs=pl.BlockSpec((2, d), lambda i: (0, 0)),
        out_shape=jax.ShapeDtypeStruct((2, d), jnp.float32),
    )(x)
    mu = out[0] / n
    var = out[1] / n - mu * mu
    return mu, var


def _fold_bn(mu, var, g, be, w, b):
    a = g * jax.lax.rsqrt(var + 1e-5)
    wf = w * a[:, None]
    bf = (be - mu * a) @ w + b
    return wf, bf[None, :]


def _mm_body(x_ref, w_ref, b_ref, o_ref, *, alpha):
    h = jnp.dot(x_ref[...], w_ref[...], preferred_element_type=jnp.float32)
    h = h + b_ref[...]
    o_ref[...] = jnp.where(h >= 0, h, alpha * h)


def _mm_stats_body(x_ref, w_ref, b_ref, o_ref, st_ref, *, alpha):
    h = jnp.dot(x_ref[...], w_ref[...], preferred_element_type=jnp.float32)
    h = h + b_ref[...]
    y = jnp.where(h >= 0, h, alpha * h)
    o_ref[...] = y

    @pl.when(pl.program_id(0) == 0)
    def _():
        st_ref[...] = jnp.zeros_like(st_ref)

    s = jnp.sum(y, axis=0, keepdims=True)
    q = jnp.sum(y * y, axis=0, keepdims=True)
    st_ref[...] += jnp.concatenate([s, q], axis=0)


def _mm_act(x, w, b, alpha, want_stats=False):
    n, din = x.shape
    dout = w.shape[1]
    bl = _pick_block(n)
    grid = (n // bl,)
    in_specs = [
        pl.BlockSpec((bl, din), lambda i: (i, 0)),
        pl.BlockSpec((din, dout), lambda i: (0, 0)),
        pl.BlockSpec((1, dout), lambda i: (0, 0)),
    ]
    if want_stats:
        y, st = pl.pallas_call(
            functools.partial(_mm_stats_body, alpha=alpha),
            grid=grid,
            in_specs=in_specs,
            out_specs=(
                pl.BlockSpec((bl, dout), lambda i: (i, 0)),
                pl.BlockSpec((2, dout), lambda i: (0, 0)),
            ),
            out_shape=(
                jax.ShapeDtypeStruct((n, dout), jnp.float32),
                jax.ShapeDtypeStruct((2, dout), jnp.float32),
            ),
        )(x, w, b)
        mu = st[0] / n
        var = st[1] / n - mu * mu
        return y, mu, var
    return pl.pallas_call(
        functools.partial(_mm_body, alpha=alpha),
        grid=grid,
        in_specs=in_specs,
        out_specs=pl.BlockSpec((bl, dout), lambda i: (i, 0)),
        out_shape=jax.ShapeDtypeStruct((n, dout), jnp.float32),
    )(x, w, b)


def _mlp_bn(x, p):
    mu1, var1 = _col_stats(x)
    w1, b1 = _fold_bn(mu1, var1, p['g1'], p['be1'], p['W1'], p['b1'])
    y, mu2, var2 = _mm_act(x, w1, b1, 0.01, want_stats=True)
    w2, b2 = _fold_bn(mu2, var2, p['g2'], p['be2'], p['W2'], p['b2'])
    return _mm_act(y, w2, b2, 0.01)


def _msg_body(xs_ref, xd_ref, e_ref, w_ref, b_ref, msg_ref, en_ref):
    h = jnp.concatenate([xs_ref[...], xd_ref[...], e_ref[...]], axis=1)
    y = jnp.dot(h, w_ref[...], preferred_element_type=jnp.float32) + b_ref[...]
    y = jnp.maximum(y, 0.0)
    msg_ref[...] = y[:, :_DIM]
    en_ref[...] = y[:, _DIM:]


def _msg_edge(xs, xd, e, wcat, bcat):
    n = xs.shape[0]
    bl = _pick_block(n)
    return pl.pallas_call(
        _msg_body,
        grid=(n // bl,),
        in_specs=[
            pl.BlockSpec((bl, _DIM), lambda i: (i, 0)),
            pl.BlockSpec((bl, _DIM), lambda i: (i, 0)),
            pl.BlockSpec((bl, _DIM), lambda i: (i, 0)),
            pl.BlockSpec((3 * _DIM, 2 * _DIM), lambda i: (0, 0)),
            pl.BlockSpec((1, 2 * _DIM), lambda i: (0, 0)),
        ],
        out_specs=(
            pl.BlockSpec((bl, _DIM), lambda i: (i, 0)),
            pl.BlockSpec((bl, _DIM), lambda i: (i, 0)),
        ),
        out_shape=(
            jax.ShapeDtypeStruct((n, _DIM), jnp.float32),
            jax.ShapeDtypeStruct((n, _DIM), jnp.float32),
        ),
    )(xs, xd, e, wcat, bcat)


def _upd_body(x_ref, a0_ref, a1_ref, c0_ref, c1_ref, w_ref, b_ref, o_ref):
    cnt = jnp.maximum(c0_ref[:, 0:1] + c1_ref[:, 0:1], 1.0)
    agg = (a0_ref[...] + a1_ref[...]) / cnt
    h = jnp.dot(agg, w_ref[...], preferred_element_type=jnp.float32)
    h = x_ref[...] + h + b_ref[...]
    o_ref[...] = jnp.maximum(h, 0.0)


def _update(x, a0, a1, c0, c1, wu, bu):
    n = x.shape[0]
    bl = _pick_block(n)
    return pl.pallas_call(
        _upd_body,
        grid=(n // bl,),
        in_specs=[
            pl.BlockSpec((bl, _DIM), lambda i: (i, 0)),
            pl.BlockSpec((bl, _DIM), lambda i: (i, 0)),
            pl.BlockSpec((bl, _DIM), lambda i: (i, 0)),
            pl.BlockSpec((bl, _DIM), lambda i: (i, 0)),
            pl.BlockSpec((bl, _DIM), lambda i: (i, 0)),
            pl.BlockSpec((_DIM, _DIM), lambda i: (0, 0)),
            pl.BlockSpec((1, _DIM), lambda i: (0, 0)),
        ],
        out_specs=pl.BlockSpec((bl, _DIM), lambda i: (i, 0)),
        out_shape=jax.ShapeDtypeStruct((n, _DIM), jnp.float32),
    )(x, a0, a1, c0, c1, wu, bu)


def _head_body(xs_ref, xd_ref, e3_ref, ea_ref, ws_ref, wd_ref, we_ref,
               wa_ref, b1_ref, w2_ref, b2_ref, o_ref):
    h = (jnp.dot(xs_ref[...], ws_ref[...], preferred_element_type=jnp.float32)
         + jnp.dot(xd_ref[...], wd_ref[...], preferred_element_type=jnp.float32)
         + jnp.dot(e3_ref[...], we_ref[...], preferred_element_type=jnp.float32)
         + jnp.dot(ea_ref[...], wa_ref[...], preferred_element_type=jnp.float32)
         + b1_ref[...])
    h = jnp.maximum(h, 0.0)
    y = jnp.dot(h, w2_ref[...], preferred_element_type=jnp.float32)
    o_ref[...] = y + b2_ref[:, 0:1]


def _head(xs, xd, e3, ea, w1, b1, w2, b2):
    n = xs.shape[0]
    bl = _pick_block(n)
    d2 = 2 * _DIM
    ea_d = ea.shape[1]
    return pl.pallas_call(
        _head_body,
        grid=(n // bl,),
        in_specs=[
            pl.BlockSpec((bl, _DIM), lambda i: (i, 0)),
            pl.BlockSpec((bl, _DIM), lambda i: (i, 0)),
            pl.BlockSpec((bl, _DIM), lambda i: (i, 0)),
            pl.BlockSpec((bl, ea_d), lambda i: (i, 0)),
            pl.BlockSpec((_DIM, d2), lambda i: (0, 0)),
            pl.BlockSpec((_DIM, d2), lambda i: (0, 0)),
            pl.BlockSpec((_DIM, d2), lambda i: (0, 0)),
            pl.BlockSpec((ea_d, d2), lambda i: (0, 0)),
            pl.BlockSpec((1, d2), lambda i: (0, 0)),
            pl.BlockSpec((d2, 1), lambda i: (0, 0)),
            pl.BlockSpec((1, _DIM), lambda i: (0, 0)),
        ],
        out_specs=pl.BlockSpec((bl, 1), lambda i: (i, 0)),
        out_shape=jax.ShapeDtypeStruct((n, 1), jnp.float32),
    )(xs, xd, e3, ea, w1[:_DIM], w1[_DIM:2 * _DIM], w1[2 * _DIM:3 * _DIM],
      w1[3 * _DIM:], b1[None, :], w2, jnp.broadcast_to(b2[None, :], (1, _DIM)))


def _sc_gather2(table, idx_s, idx_d):
    epad = idx_s.shape[0]
    per_w = epad // _NW
    nch = per_w // _CH
    njj = nch // 2
    mesh = plsc.VectorSubcoreMesh(core_axis_name="c", subcore_axis_name="s")

    @functools.partial(
        pl.kernel, mesh=mesh,
        out_type=(jax.ShapeDtypeStruct((epad, _DIM), jnp.float32),
                  jax.ShapeDtypeStruct((epad, _DIM), jnp.float32)),
        scratch_types=[
            pltpu.VMEM((per_w,), jnp.int32),
            pltpu.VMEM((per_w,), jnp.int32),
            pltpu.VMEM((_CH, _DIM), jnp.float32),
            pltpu.VMEM((_CH, _DIM), jnp.float32),
            pltpu.VMEM((_CH, _DIM), jnp.float32),
            pltpu.VMEM((_CH, _DIM), jnp.float32),
        ] + [pltpu.SemaphoreType.DMA] * 8,
    )
    def k(table_hbm, idxs_hbm, idxd_hbm, outs_hbm, outd_hbm,
          idxs_v, idxd_v, bs0, bs1, bd0, bd1,
          gs0, gs1, gd0, gd1, ws0, ws1, wd0, wd1):
        wid = lax.axis_index("s") * _NC + lax.axis_index("c")
        base = pl.multiple_of(wid * per_w, 8)
        pltpu.sync_copy(idxs_hbm.at[pl.ds(base, per_w)], idxs_v)
        pltpu.sync_copy(idxd_hbm.at[pl.ds(base, per_w)], idxd_v)

        def _ichunk(ref, c):
            return ref.at[pl.ds(pl.multiple_of(c * _CH, 8), _CH)]

        def _orow(c):
            return pl.ds(pl.multiple_of(base + c * _CH, 8), _CH)

        pltpu.async_copy(table_hbm.at[_ichunk(idxs_v, 0)], bs0, gs0)
        pltpu.async_copy(table_hbm.at[_ichunk(idxd_v, 0)], bd0, gd0)

        def body(j, _):
            e = 2 * j
            o = e + 1
            pltpu.make_async_copy(table_hbm.at[_ichunk(idxs_v, e)], bs0, gs0).wait()
            pltpu.make_async_copy(table_hbm.at[_ichunk(idxd_v, e)], bd0, gd0).wait()

            @pl.when(j > 0)
            def _():
                pltpu.make_async_copy(bs1, outs_hbm.at[_orow(o - 2)], ws1).wait()
                pltpu.make_async_copy(bd1, outd_hbm.at[_orow(o - 2)], wd1).wait()

            pltpu.async_copy(table_hbm.at[_ichunk(idxs_v, o)], bs1, gs1)
            pltpu.async_copy(table_hbm.at[_ichunk(idxd_v, o)], bd1, gd1)
            pltpu.async_copy(bs0, outs_hbm.at[_orow(e)], ws0)
            pltpu.async_copy(bd0, outd_hbm.at[_orow(e)], wd0)

            pltpu.make_async_copy(table_hbm.at[_ichunk(idxs_v, o)], bs1, gs1).wait()
            pltpu.make_async_copy(table_hbm.at[_ichunk(idxd_v, o)], bd1, gd1).wait()
            pltpu.make_async_copy(bs0, outs_hbm.at[_orow(e)], ws0).wait()
            pltpu.make_async_copy(bd0, outd_hbm.at[_orow(e)], wd0).wait()

            @pl.when(j + 1 < njj)
            def _():
                pltpu.async_copy(table_hbm.at[_ichunk(idxs_v, e + 2)], bs0, gs0)
                pltpu.async_copy(table_hbm.at[_ichunk(idxd_v, e + 2)], bd0, gd0)

            pltpu.async_copy(bs1, outs_hbm.at[_orow(o)], ws1)
            pltpu.async_copy(bd1, outd_hbm.at[_orow(o)], wd1)
            return 0

        lax.fori_loop(0, njj, body, 0, unroll=False)
        pltpu.make_async_copy(bs1, outs_hbm.at[_orow(nch - 1)], ws1).wait()
        pltpu.make_async_copy(bd1, outd_hbm.at[_orow(nch - 1)], wd1).wait()

    return k(table, idx_s, idx_d)


def _sc_scatter_add(msg, idx, zeros_acc):
    epad = idx.shape[0]
    mesh = plsc.VectorSubcoreMesh(core_axis_name="c", subcore_axis_name="s")
    rows_per_tile = _NPAD // _NS

    per_w = epad // _NW
    nch = per_w // _CH
    njj = nch // 2

    @functools.partial(
        pl.kernel, mesh=mesh,
        out_type=jax.ShapeDtypeStruct((_NC, _NPAD, _DIM), jnp.float32),
        scratch_types=[
            pltpu.VMEM((2, _CH), jnp.int32),
            pltpu.VMEM((_CH, _DIM), jnp.float32),
            pltpu.VMEM((_CH, _DIM), jnp.float32),
            pltpu.VMEM_SHARED((_NPAD, _DIM), jnp.float32),
            pltpu.SemaphoreType.DMA,
            pltpu.SemaphoreType.DMA,
            pltpu.SemaphoreType.DMA,
            pltpu.SemaphoreType.DMA,
        ],
    )
    def k(msg_hbm, idx_hbm, z_hbm, out_hbm, i2, m0, m1, acc, l0, l1, i0, i1):
        c = lax.axis_index("c")
        s = lax.axis_index("s")
        wid = s * _NC + c
        base = pl.multiple_of(wid * per_w, 8)

        @pl.when(s == 0)
        def _():
            pltpu.sync_copy(z_hbm, acc)

        def _row(ch):
            return pl.ds(pl.multiple_of(base + ch * _CH, 8), _CH)

        plsc.subcore_barrier()
        pltpu.async_copy(msg_hbm.at[_row(0)], m0, l0)
        pltpu.async_copy(idx_hbm.at[_row(0)], i2.at[0], i0)

        def body(j, _):
            e = 2 * j
            o = e + 1
            pltpu.make_async_copy(msg_hbm.at[_row(e)], m0, l0).wait()
            pltpu.make_async_copy(idx_hbm.at[_row(e)], i2.at[0], i0).wait()
            pltpu.async_copy(msg_hbm.at[_row(o)], m1, l1)
            pltpu.async_copy(idx_hbm.at[_row(o)], i2.at[1], i1)
            pltpu.sync_copy(m0, acc.at[i2.at[0]], add=True)
            pltpu.make_async_copy(msg_hbm.at[_row(o)], m1, l1).wait()
            pltpu.make_async_copy(idx_hbm.at[_row(o)], i2.at[1], i1).wait()

            @pl.when(j + 1 < njj)
            def _():
                pltpu.async_copy(msg_hbm.at[_row(e + 2)], m0, l0)
                pltpu.async_copy(idx_hbm.at[_row(e + 2)], i2.at[0], i0)

            pltpu.sync_copy(m1, acc.at[i2.at[1]], add=True)
            return 0

        lax.fori_loop(0, njj, body, 0, unroll=False)
        plsc.subcore_barrier()
        orow = pl.ds(pl.multiple_of(s * rows_per_tile, 8), rows_per_tile)
        pltpu.sync_copy(acc.at[orow], out_hbm.at[c, orow])

    return k(msg, idx, zeros_acc)


def _sc_count(idx, ones_chunk, zeros_cnt):
    epad = idx.shape[0]
    per_w = epad // _NW
    nch = per_w // _CH
    njj = nch // 2
    mesh = plsc.VectorSubcoreMesh(core_axis_name="c", subcore_axis_name="s")
    rows_per_tile = _NPAD // _NS

    @functools.partial(
        pl.kernel, mesh=mesh,
        out_type=jax.ShapeDtypeStruct((_NC, _NPAD, _DIM), jnp.float32),
        scratch_types=[
            pltpu.VMEM((2, _CH), jnp.int32),
            pltpu.VMEM((_CH, _DIM), jnp.float32),
            pltpu.VMEM_SHARED((_NPAD, _DIM), jnp.float32),
            pltpu.SemaphoreType.DMA,
            pltpu.SemaphoreType.DMA,
        ],
    )
    def k(idx_hbm, ones_hbm, z_hbm, out_hbm, i2, ones_v, acc, i0, i1):
        c = lax.axis_index("c")
        s = lax.axis_index("s")
        wid = s * _NC + c
        base = pl.multiple_of(wid * per_w, 8)

        @pl.when(s == 0)
        def _():
            pltpu.sync_copy(z_hbm, acc)

        def _row(ch):
            return pl.ds(pl.multiple_of(base + ch * _CH, 8), _CH)

        pltpu.sync_copy(ones_hbm, ones_v)
        plsc.subcore_barrier()
        pltpu.async_copy(idx_hbm.at[_row(0)], i2.at[0], i0)

        def body(j, _):
            e = 2 * j
            o = e + 1
            pltpu.make_async_copy(idx_hbm.at[_row(e)], i2.at[0], i0).wait()
            pltpu.async_copy(idx_hbm.at[_row(o)], i2.at[1], i1)
            pltpu.sync_copy(ones_v, acc.at[i2.at[0]], add=True)
            pltpu.make_async_copy(idx_hbm.at[_row(o)], i2.at[1], i1).wait()

            @pl.when(j + 1 < njj)
            def _():
                pltpu.async_copy(idx_hbm.at[_row(e + 2)], i2.at[0], i0)

            pltpu.sync_copy(ones_v, acc.at[i2.at[1]], add=True)
            return 0

        lax.fori_loop(0, njj, body, 0, unroll=False)
        plsc.subcore_barrier()
        orow = pl.ds(pl.multiple_of(s * rows_per_tile, 8), rows_per_tile)
        pltpu.sync_copy(acc.at[orow], out_hbm.at[c, orow])

    return k(idx, ones_chunk, zeros_cnt)


def _pad_to(a, n, fill=0):
    pad = n - a.shape[0]
    if pad == 0:
        return a
    shape = (pad,) + a.shape[1:]
    return jnp.concatenate([a, jnp.full(shape, fill, a.dtype)], axis=0)


def _round_up(n, m):
    return ((n + m - 1) // m) * m


def kernel(x, edge_attr, edge_attr3, edge_attr4, params, edge_index, edge_index3):
    n_e = edge_index.shape[1]
    n_e3 = edge_index3.shape[1]
    ep1 = _round_up(n_e, 2 * _NW * _CH)
    ep3 = _round_up(2 * n_e3, 2 * _NW * _CH)
    eph = _round_up(n_e3, 2 * _NW * _CH)

    zeros_acc = jnp.zeros((_NPAD, _DIM), jnp.float32)
    ones_chunk = jnp.ones((_CH, _DIM), jnp.float32)

    src1 = _pad_to(edge_index[0], ep1)
    dst1 = _pad_to(edge_index[1], ep1)
    dst1_sc = _pad_to(edge_index[1], ep1, _DUMMY)

    ei3_full = jnp.concatenate([edge_index3, edge_index3[jnp.array([1, 0])]],
                               axis=1)
    src3 = _pad_to(ei3_full[0], ep3)
    dst3 = _pad_to(ei3_full[1], ep3)
    dst3_sc = _pad_to(ei3_full[1], ep3, _DUMMY)

    srch = _pad_to(edge_index3[0], eph)
    dsth = _pad_to(edge_index3[1], eph)

    out = _mlp_bn(x, params['node'])
    e = _pad_to(_mlp_bn(edge_attr, params['edge1']), ep1)
    temp = _mlp_bn(jnp.concatenate([edge_attr3, edge_attr4], axis=1),
                   params['edge2'])
    e3 = _pad_to(jnp.concatenate([temp, temp], axis=0), ep3)

    cnt1 = _sc_count(dst1_sc, ones_chunk, zeros_acc)
    c1a, c1b = cnt1[0, :_N], cnt1[1, :_N]
    cnt3 = _sc_count(dst3_sc, ones_chunk, zeros_acc)
    c3a, c3b = cnt3[0, :_N], cnt3[1, :_N]

    for p in params['conv1']:
        wcat = jnp.concatenate([p['Wm'], p['We']], axis=1)
        bcat = jnp.concatenate([p['bm'], p['be']])[None, :]
        xs, xd = _sc_gather2(out, src1, dst1)
        msg, e = _msg_edge(xs, xd, e, wcat, bcat)
        parts = _sc_scatter_add(msg, dst1_sc, zeros_acc)
        out = _update(out, parts[0, :_N], parts[1, :_N], c1a, c1b,
                      p['Wu'], p['bu'][None, :])

    for p in params['conv2']:
        wcat = jnp.concatenate([p['Wm'], p['We']], axis=1)
        bcat = jnp.concatenate([p['bm'], p['be']])[None, :]
        xs, xd = _sc_gather2(out, src3, dst3)
        msg, e3 = _msg_edge(xs, xd, e3, wcat, bcat)
        parts = _sc_scatter_add(msg, dst3_sc, zeros_acc)
        out = _update(out, parts[0, :_N], parts[1, :_N], c3a, c3b,
                      p['Wu'], p['bu'][None, :])

    xs, xd = _sc_gather2(out, srch, dsth)
    hp = params['head']
    yh = _head(xs, xd, e3[:eph], _pad_to(edge_attr3, eph),
               hp['W1'], hp['b1'], hp['W2'], hp['b2'])
    return yh[:n_e3, 0]


# speedup vs baseline: 1.0926x; 1.0926x over previous
import functools

import jax
import jax.numpy as jnp
from jax import lax
from jax.experimental import pallas as pl
from jax.experimental.pallas import tpu as pltpu
from jax.experimental.pallas import tpu_sc as plsc

_DIM = 128
_N = 10000
_NPAD = 10240
_DUMMY = 10000
_NC, _NS = 2, 16
_NW = _NC * _NS
_CH = 128


def _pick_block(n):
    for b in (512, 400, 320, 256, 200, 160, 128, 80, 64, 40, 32, 16, 8):
        if n % b == 0:
            return b
    raise ValueError(f"no row block for {n}")


def _stats_body(x_ref, o_ref):
    xb = x_ref[...]
    s = jnp.sum(xb, axis=0, keepdims=True)
    q = jnp.sum(xb * xb, axis=0, keepdims=True)

    @pl.when(pl.program_id(0) == 0)
    def _():
        o_ref[...] = jnp.zeros_like(o_ref)

    o_ref[...] += jnp.concatenate([s, q], axis=0)


def _col_stats(x):
    n, d = x.shape
    b = _pick_block(n)
    out = pl.pallas_call(
        _stats_body,
        grid=(n // b,),
        in_specs=[pl.BlockSpec((b, d), lambda i: (i, 0))],
        out_specs=pl.BlockSpec((2, d), lambda i: (0, 0)),
        out_shape=jax.ShapeDtypeStruct((2, d), jnp.float32),
    )(x)
    mu = out[0] / n
    var = out[1] / n - mu * mu
    return mu, var


def _fold_bn(mu, var, g, be, w, b):
    a = g * jax.lax.rsqrt(var + 1e-5)
    wf = w * a[:, None]
    bf = (be - mu * a) @ w + b
    return wf, bf[None, :]


def _mm_body(x_ref, w_ref, b_ref, o_ref, *, alpha):
    h = jnp.dot(x_ref[...], w_ref[...], preferred_element_type=jnp.float32)
    h = h + b_ref[...]
    o_ref[...] = jnp.where(h >= 0, h, alpha * h)


def _mm_stats_body(x_ref, w_ref, b_ref, o_ref, st_ref, *, alpha):
    h = jnp.dot(x_ref[...], w_ref[...], preferred_element_type=jnp.float32)
    h = h + b_ref[...]
    y = jnp.where(h >= 0, h, alpha * h)
    o_ref[...] = y

    @pl.when(pl.program_id(0) == 0)
    def _():
        st_ref[...] = jnp.zeros_like(st_ref)

    s = jnp.sum(y, axis=0, keepdims=True)
    q = jnp.sum(y * y, axis=0, keepdims=True)
    st_ref[...] += jnp.concatenate([s, q], axis=0)


def _mm_act(x, w, b, alpha, want_stats=False):
    n, din = x.shape
    dout = w.shape[1]
    bl = _pick_block(n)
    grid = (n // bl,)
    in_specs = [
        pl.BlockSpec((bl, din), lambda i: (i, 0)),
        pl.BlockSpec((din, dout), lambda i: (0, 0)),
        pl.BlockSpec((1, dout), lambda i: (0, 0)),
    ]
    if want_stats:
        y, st = pl.pallas_call(
            functools.partial(_mm_stats_body, alpha=alpha),
            grid=grid,
            in_specs=in_specs,
            out_specs=(
                pl.BlockSpec((bl, dout), lambda i: (i, 0)),
                pl.BlockSpec((2, dout), lambda i: (0, 0)),
            ),
            out_shape=(
                jax.ShapeDtypeStruct((n, dout), jnp.float32),
                jax.ShapeDtypeStruct((2, dout), jnp.float32),
            ),
        )(x, w, b)
        mu = st[0] / n
        var = st[1] / n - mu * mu
        return y, mu, var
    return pl.pallas_call(
        functools.partial(_mm_body, alpha=alpha),
        grid=grid,
        in_specs=in_specs,
        out_specs=pl.BlockSpec((bl, dout), lambda i: (i, 0)),
        out_shape=jax.ShapeDtypeStruct((n, dout), jnp.float32),
    )(x, w, b)


def _mlp_bn(x, p):
    mu1, var1 = _col_stats(x)
    w1, b1 = _fold_bn(mu1, var1, p['g1'], p['be1'], p['W1'], p['b1'])
    y, mu2, var2 = _mm_act(x, w1, b1, 0.01, want_stats=True)
    w2, b2 = _fold_bn(mu2, var2, p['g2'], p['be2'], p['W2'], p['b2'])
    return _mm_act(y, w2, b2, 0.01)


def _msg_body(xs_ref, xd_ref, e_ref, w_ref, b_ref, msg_ref, en_ref):
    h = jnp.concatenate([xs_ref[...], xd_ref[...], e_ref[...]], axis=1)
    y = jnp.dot(h, w_ref[...], preferred_element_type=jnp.float32) + b_ref[...]
    y = jnp.maximum(y, 0.0)
    msg_ref[...] = y[:, :_DIM]
    en_ref[...] = y[:, _DIM:]


def _msg_edge(xs, xd, e, wcat, bcat):
    n = xs.shape[0]
    bl = _pick_block(n)
    return pl.pallas_call(
        _msg_body,
        grid=(n // bl,),
        in_specs=[
            pl.BlockSpec((bl, _DIM), lambda i: (i, 0)),
            pl.BlockSpec((bl, _DIM), lambda i: (i, 0)),
            pl.BlockSpec((bl, _DIM), lambda i: (i, 0)),
            pl.BlockSpec((3 * _DIM, 2 * _DIM), lambda i: (0, 0)),
            pl.BlockSpec((1, 2 * _DIM), lambda i: (0, 0)),
        ],
        out_specs=(
            pl.BlockSpec((bl, _DIM), lambda i: (i, 0)),
            pl.BlockSpec((bl, _DIM), lambda i: (i, 0)),
        ),
        out_shape=(
            jax.ShapeDtypeStruct((n, _DIM), jnp.float32),
            jax.ShapeDtypeStruct((n, _DIM), jnp.float32),
        ),
    )(xs, xd, e, wcat, bcat)


def _msg2_body(gs_ref, gd_ref, ea_ref, eb_ref, w_ref, b_ref,
               mf_ref, mb_ref, ef_ref, eb2_ref):
    xs = gs_ref[...]
    xd = gd_ref[...]
    w = w_ref[...]
    b = b_ref[...]
    hf = jnp.concatenate([xs, xd, ea_ref[...]], axis=1)
    yf = jnp.maximum(jnp.dot(hf, w, preferred_element_type=jnp.float32) + b, 0.0)
    mf_ref[...] = yf[:, :_DIM]
    ef_ref[...] = yf[:, _DIM:]
    hb = jnp.concatenate([xd, xs, eb_ref[...]], axis=1)
    yb = jnp.maximum(jnp.dot(hb, w, preferred_element_type=jnp.float32) + b, 0.0)
    mb_ref[...] = yb[:, :_DIM]
    eb2_ref[...] = yb[:, _DIM:]


def _msg_edge2(gs, gd, ea, eb, wcat, bcat):
    n = gs.shape[0]
    bl = _pick_block(n)
    rspec = pl.BlockSpec((bl, _DIM), lambda i: (i, 0))
    return pl.pallas_call(
        _msg2_body,
        grid=(n // bl,),
        in_specs=[
            rspec, rspec, rspec, rspec,
            pl.BlockSpec((3 * _DIM, 2 * _DIM), lambda i: (0, 0)),
            pl.BlockSpec((1, 2 * _DIM), lambda i: (0, 0)),
        ],
        out_specs=(rspec, rspec, rspec, rspec),
        out_shape=tuple(jax.ShapeDtypeStruct((n, _DIM), jnp.float32)
                        for _ in range(4)),
    )(gs, gd, ea, eb, wcat, bcat)


def _upd_body(x_ref, a0_ref, a1_ref, c0_ref, c1_ref, w_ref, b_ref, o_ref):
    cnt = jnp.maximum(c0_ref[:, 0:1] + c1_ref[:, 0:1], 1.0)
    agg = (a0_ref[...] + a1_ref[...]) / cnt
    h = jnp.dot(agg, w_ref[...], preferred_element_type=jnp.float32)
    h = x_ref[...] + h + b_ref[...]
    o_ref[...] = jnp.maximum(h, 0.0)


def _update(x, a0, a1, c0, c1, wu, bu):
    n = x.shape[0]
    bl = _pick_block(n)
    return pl.pallas_call(
        _upd_body,
        grid=(n // bl,),
        in_specs=[
            pl.BlockSpec((bl, _DIM), lambda i: (i, 0)),
            pl.BlockSpec((bl, _DIM), lambda i: (i, 0)),
            pl.BlockSpec((bl, _DIM), lambda i: (i, 0)),
            pl.BlockSpec((bl, _DIM), lambda i: (i, 0)),
            pl.BlockSpec((bl, _DIM), lambda i: (i, 0)),
            pl.BlockSpec((_DIM, _DIM), lambda i: (0, 0)),
            pl.BlockSpec((1, _DIM), lambda i: (0, 0)),
        ],
        out_specs=pl.BlockSpec((bl, _DIM), lambda i: (i, 0)),
        out_shape=jax.ShapeDtypeStruct((n, _DIM), jnp.float32),
    )(x, a0, a1, c0, c1, wu, bu)


def _head_body(xs_ref, xd_ref, e3_ref, ea_ref, ws_ref, wd_ref, we_ref,
               wa_ref, b1_ref, w2_ref, b2_ref, o_ref):
    h = (jnp.dot(xs_ref[...], ws_ref[...], preferred_element_type=jnp.float32)
         + jnp.dot(xd_ref[...], wd_ref[...], preferred_element_type=jnp.float32)
         + jnp.dot(e3_ref[...], we_ref[...], preferred_element_type=jnp.float32)
         + jnp.dot(ea_ref[...], wa_ref[...], preferred_element_type=jnp.float32)
         + b1_ref[...])
    h = jnp.maximum(h, 0.0)
    y = jnp.dot(h, w2_ref[...], preferred_element_type=jnp.float32)
    o_ref[...] = y + b2_ref[:, 0:1]


def _head(xs, xd, e3, ea, w1, b1, w2, b2):
    n = xs.shape[0]
    bl = _pick_block(n)
    d2 = 2 * _DIM
    ea_d = ea.shape[1]
    return pl.pallas_call(
        _head_body,
        grid=(n // bl,),
        in_specs=[
            pl.BlockSpec((bl, _DIM), lambda i: (i, 0)),
            pl.BlockSpec((bl, _DIM), lambda i: (i, 0)),
            pl.BlockSpec((bl, _DIM), lambda i: (i, 0)),
            pl.BlockSpec((bl, ea_d), lambda i: (i, 0)),
            pl.BlockSpec((_DIM, d2), lambda i: (0, 0)),
            pl.BlockSpec((_DIM, d2), lambda i: (0, 0)),
            pl.BlockSpec((_DIM, d2), lambda i: (0, 0)),
            pl.BlockSpec((ea_d, d2), lambda i: (0, 0)),
            pl.BlockSpec((1, d2), lambda i: (0, 0)),
            pl.BlockSpec((d2, 1), lambda i: (0, 0)),
            pl.BlockSpec((1, _DIM), lambda i: (0, 0)),
        ],
        out_specs=pl.BlockSpec((bl, 1), lambda i: (i, 0)),
        out_shape=jax.ShapeDtypeStruct((n, 1), jnp.float32),
    )(xs, xd, e3, ea, w1[:_DIM], w1[_DIM:2 * _DIM], w1[2 * _DIM:3 * _DIM],
      w1[3 * _DIM:], b1[None, :], w2, jnp.broadcast_to(b2[None, :], (1, _DIM)))


def _sc_gather2(table, idx_s, idx_d):
    epad = idx_s.shape[0]
    per_w = epad // _NW
    nch = per_w // _CH
    njj = nch // 2
    mesh = plsc.VectorSubcoreMesh(core_axis_name="c", subcore_axis_name="s")

    @functools.partial(
        pl.kernel, mesh=mesh,
        out_type=(jax.ShapeDtypeStruct((epad, _DIM), jnp.float32),
                  jax.ShapeDtypeStruct((epad, _DIM), jnp.float32)),
        scratch_types=[
            pltpu.VMEM((_CH,), jnp.int32),
            pltpu.VMEM((_CH,), jnp.int32),
            pltpu.VMEM((_CH,), jnp.int32),
            pltpu.VMEM((_CH,), jnp.int32),
            pltpu.VMEM((_CH, _DIM), jnp.float32),
            pltpu.VMEM((_CH, _DIM), jnp.float32),
            pltpu.VMEM((_CH, _DIM), jnp.float32),
            pltpu.VMEM((_CH, _DIM), jnp.float32),
        ] + [pltpu.SemaphoreType.DMA] * 4,
    )
    def k(table_hbm, idxs_hbm, idxd_hbm, outs_hbm, outd_hbm,
          is0, is1, id0, id1, bs0, bs1, bd0, bd1,
          gs0, gs1, gd0, gd1):
        wid = lax.axis_index("s") * _NC + lax.axis_index("c")
        base = pl.multiple_of(wid * per_w, 8)

        def _irow(c):
            return pl.ds(pl.multiple_of(base + c * _CH, 8), _CH)

        pltpu.sync_copy(idxs_hbm.at[_irow(0)], is0)
        pltpu.sync_copy(idxd_hbm.at[_irow(0)], id0)
        pltpu.async_copy(table_hbm.at[is0], bs0, gs0)
        pltpu.async_copy(table_hbm.at[id0], bd0, gd0)

        def body(j, _):
            e = 2 * j
            o = e + 1
            pltpu.sync_copy(idxs_hbm.at[_irow(o)], is1)
            pltpu.sync_copy(idxd_hbm.at[_irow(o)], id1)
            pltpu.async_copy(table_hbm.at[is1], bs1, gs1)
            pltpu.async_copy(table_hbm.at[id1], bd1, gd1)
            pltpu.make_async_copy(table_hbm.at[is0], bs0, gs0).wait()
            pltpu.make_async_copy(table_hbm.at[id0], bd0, gd0).wait()
            pltpu.sync_copy(bs0, outs_hbm.at[_irow(e)])
            pltpu.sync_copy(bd0, outd_hbm.at[_irow(e)])

            @pl.when(j + 1 < njj)
            def _():
                pltpu.sync_copy(idxs_hbm.at[_irow(e + 2)], is0)
                pltpu.sync_copy(idxd_hbm.at[_irow(e + 2)], id0)
                pltpu.async_copy(table_hbm.at[is0], bs0, gs0)
                pltpu.async_copy(table_hbm.at[id0], bd0, gd0)

            pltpu.make_async_copy(table_hbm.at[is1], bs1, gs1).wait()
            pltpu.make_async_copy(table_hbm.at[id1], bd1, gd1).wait()
            pltpu.sync_copy(bs1, outs_hbm.at[_irow(o)])
            pltpu.sync_copy(bd1, outd_hbm.at[_irow(o)])
            return 0

        lax.fori_loop(0, njj, body, 0, unroll=False)

    return k(table, idx_s, idx_d)


def _sc_scatter_add(msg, idx, zeros_acc):
    epad = idx.shape[0]
    mesh = plsc.VectorSubcoreMesh(core_axis_name="c", subcore_axis_name="s")
    rows_per_tile = _NPAD // _NS

    per_w = epad // _NW
    nch = per_w // _CH
    njj = nch // 2

    @functools.partial(
        pl.kernel, mesh=mesh,
        out_type=jax.ShapeDtypeStruct((_NC, _NPAD, _DIM), jnp.float32),
        scratch_types=[
            pltpu.VMEM((2, _CH), jnp.int32),
            pltpu.VMEM((_CH, _DIM), jnp.float32),
            pltpu.VMEM((_CH, _DIM), jnp.float32),
            pltpu.VMEM_SHARED((_NPAD, _DIM), jnp.float32),
            pltpu.SemaphoreType.DMA,
            pltpu.SemaphoreType.DMA,
            pltpu.SemaphoreType.DMA,
            pltpu.SemaphoreType.DMA,
        ],
    )
    def k(msg_hbm, idx_hbm, z_hbm, out_hbm, i2, m0, m1, acc, l0, l1, i0, i1):
        c = lax.axis_index("c")
        s = lax.axis_index("s")
        wid = s * _NC + c
        base = pl.multiple_of(wid * per_w, 8)

        @pl.when(s == 0)
        def _():
            pltpu.sync_copy(z_hbm, acc)

        def _row(ch):
            return pl.ds(pl.multiple_of(base + ch * _CH, 8), _CH)

        plsc.subcore_barrier()
        pltpu.async_copy(msg_hbm.at[_row(0)], m0, l0)
        pltpu.async_copy(idx_hbm.at[_row(0)], i2.at[0], i0)

        def body(j, _):
            e = 2 * j
            o = e + 1
            pltpu.make_async_copy(msg_hbm.at[_row(e)], m0, l0).wait()
            pltpu.make_async_copy(idx_hbm.at[_row(e)], i2.at[0], i0).wait()
            pltpu.async_copy(msg_hbm.at[_row(o)], m1, l1)
            pltpu.async_copy(idx_hbm.at[_row(o)], i2.at[1], i1)
            pltpu.sync_copy(m0, acc.at[i2.at[0]], add=True)
            pltpu.make_async_copy(msg_hbm.at[_row(o)], m1, l1).wait()
            pltpu.make_async_copy(idx_hbm.at[_row(o)], i2.at[1], i1).wait()

            @pl.when(j + 1 < njj)
            def _():
                pltpu.async_copy(msg_hbm.at[_row(e + 2)], m0, l0)
                pltpu.async_copy(idx_hbm.at[_row(e + 2)], i2.at[0], i0)

            pltpu.sync_copy(m1, acc.at[i2.at[1]], add=True)
            return 0

        lax.fori_loop(0, njj, body, 0, unroll=False)
        plsc.subcore_barrier()
        orow = pl.ds(pl.multiple_of(s * rows_per_tile, 8), rows_per_tile)
        pltpu.sync_copy(acc.at[orow], out_hbm.at[c, orow])

    return k(msg, idx, zeros_acc)


def _sc_scatter_add2(msg_a, msg_b, idx_a, idx_b, zeros_acc):
    epad = idx_a.shape[0]
    mesh = plsc.VectorSubcoreMesh(core_axis_name="c", subcore_axis_name="s")
    rows_per_tile = _NPAD // _NS
    per_w = epad // _NW
    nch = per_w // _CH
    njj = nch // 2

    @functools.partial(
        pl.kernel, mesh=mesh,
        out_type=jax.ShapeDtypeStruct((_NC, _NPAD, _DIM), jnp.float32),
        scratch_types=[
            pltpu.VMEM((2, _CH), jnp.int32),
            pltpu.VMEM((_CH, _DIM), jnp.float32),
            pltpu.VMEM((_CH, _DIM), jnp.float32),
            pltpu.VMEM_SHARED((_NPAD, _DIM), jnp.float32),
            pltpu.SemaphoreType.DMA,
            pltpu.SemaphoreType.DMA,
            pltpu.SemaphoreType.DMA,
            pltpu.SemaphoreType.DMA,
        ],
    )
    def k(msga_hbm, msgb_hbm, idxa_hbm, idxb_hbm, z_hbm, out_hbm,
          i2, m0, m1, acc, l0, l1, i0, i1):
        c = lax.axis_index("c")
        s = lax.axis_index("s")
        wid = s * _NC + c
        base = pl.multiple_of(wid * per_w, 8)

        @pl.when(s == 0)
        def _():
            pltpu.sync_copy(z_hbm, acc)

        def _row(ch):
            return pl.ds(pl.multiple_of(base + ch * _CH, 8), _CH)

        plsc.subcore_barrier()

        def _pass(msg_hbm, idx_hbm):
            pltpu.async_copy(msg_hbm.at[_row(0)], m0, l0)
            pltpu.async_copy(idx_hbm.at[_row(0)], i2.at[0], i0)

            def body(j, _):
                e = 2 * j
                o = e + 1
                pltpu.make_async_copy(msg_hbm.at[_row(e)], m0, l0).wait()
                pltpu.make_async_copy(idx_hbm.at[_row(e)], i2.at[0], i0).wait()
                pltpu.async_copy(msg_hbm.at[_row(o)], m1, l1)
                pltpu.async_copy(idx_hbm.at[_row(o)], i2.at[1], i1)
                pltpu.sync_copy(m0, acc.at[i2.at[0]], add=True)
                pltpu.make_async_copy(msg_hbm.at[_row(o)], m1, l1).wait()
                pltpu.make_async_copy(idx_hbm.at[_row(o)], i2.at[1], i1).wait()

                @pl.when(j + 1 < njj)
                def _():
                    pltpu.async_copy(msg_hbm.at[_row(e + 2)], m0, l0)
                    pltpu.async_copy(idx_hbm.at[_row(e + 2)], i2.at[0], i0)

                pltpu.sync_copy(m1, acc.at[i2.at[1]], add=True)
                return 0

            lax.fori_loop(0, njj, body, 0, unroll=False)

        _pass(msga_hbm, idxa_hbm)
        _pass(msgb_hbm, idxb_hbm)
        plsc.subcore_barrier()
        orow = pl.ds(pl.multiple_of(s * rows_per_tile, 8), rows_per_tile)
        pltpu.sync_copy(acc.at[orow], out_hbm.at[c, orow])

    return k(msg_a, msg_b, idx_a, idx_b, zeros_acc)


def _sc_count(idxs, ones_chunk, zeros_cnt):
    idxs = tuple(idxs)
    epad = idxs[0].shape[0]
    per_w = epad // _NW
    nch = per_w // _CH
    njj = nch // 2
    mesh = plsc.VectorSubcoreMesh(core_axis_name="c", subcore_axis_name="s")
    rows_per_tile = _NPAD // _NS

    @functools.partial(
        pl.kernel, mesh=mesh,
        out_type=jax.ShapeDtypeStruct((_NC, _NPAD, _DIM), jnp.float32),
        scratch_types=[
            pltpu.VMEM((2, _CH), jnp.int32),
            pltpu.VMEM((_CH, _DIM), jnp.float32),
            pltpu.VMEM_SHARED((_NPAD, _DIM), jnp.float32),
            pltpu.SemaphoreType.DMA,
            pltpu.SemaphoreType.DMA,
        ],
    )
    def k(*refs):
        idx_hbms = refs[:len(idxs)]
        ones_hbm, z_hbm, out_hbm, i2, ones_v, acc, i0, i1 = refs[len(idxs):]
        c = lax.axis_index("c")
        s = lax.axis_index("s")
        wid = s * _NC + c
        base = pl.multiple_of(wid * per_w, 8)

        @pl.when(s == 0)
        def _():
            pltpu.sync_copy(z_hbm, acc)

        def _row(ch):
            return pl.ds(pl.multiple_of(base + ch * _CH, 8), _CH)

        pltpu.sync_copy(ones_hbm, ones_v)
        plsc.subcore_barrier()

        def _pass(idx_hbm):
            pltpu.async_copy(idx_hbm.at[_row(0)], i2.at[0], i0)

            def body(j, _):
                e = 2 * j
                o = e + 1
                pltpu.make_async_copy(idx_hbm.at[_row(e)], i2.at[0], i0).wait()
                pltpu.async_copy(idx_hbm.at[_row(o)], i2.at[1], i1)
                pltpu.sync_copy(ones_v, acc.at[i2.at[0]], add=True)
                pltpu.make_async_copy(idx_hbm.at[_row(o)], i2.at[1], i1).wait()

                @pl.when(j + 1 < njj)
                def _():
                    pltpu.async_copy(idx_hbm.at[_row(e + 2)], i2.at[0], i0)

                pltpu.sync_copy(ones_v, acc.at[i2.at[1]], add=True)
                return 0

            lax.fori_loop(0, njj, body, 0, unroll=False)

        for idx_hbm in idx_hbms:
            _pass(idx_hbm)
        plsc.subcore_barrier()
        orow = pl.ds(pl.multiple_of(s * rows_per_tile, 8), rows_per_tile)
        pltpu.sync_copy(acc.at[orow], out_hbm.at[c, orow])

    return k(*idxs, ones_chunk, zeros_cnt)


def _pad_to(a, n, fill=0):
    pad = n - a.shape[0]
    if pad == 0:
        return a
    shape = (pad,) + a.shape[1:]
    return jnp.concatenate([a, jnp.full(shape, fill, a.dtype)], axis=0)


def _round_up(n, m):
    return ((n + m - 1) // m) * m


def kernel(x, edge_attr, edge_attr3, edge_attr4, params, edge_index, edge_index3):
    n_e = edge_index.shape[1]
    n_e3 = edge_index3.shape[1]
    ep1 = _round_up(n_e, 2 * _NW * _CH)
    eph = _round_up(n_e3, 2 * _NW * _CH)

    zeros_acc = jnp.zeros((_NPAD, _DIM), jnp.float32)
    ones_chunk = jnp.ones((_CH, _DIM), jnp.float32)

    src1 = _pad_to(edge_index[0], ep1)
    dst1 = _pad_to(edge_index[1], ep1)
    dst1_sc = _pad_to(edge_index[1], ep1, _DUMMY)

    src3 = _pad_to(edge_index3[0], eph)
    dst3 = _pad_to(edge_index3[1], eph)
    src3_sc = _pad_to(edge_index3[0], eph, _DUMMY)
    dst3_sc = _pad_to(edge_index3[1], eph, _DUMMY)

    out = _mlp_bn(x, params['node'])
    e = _pad_to(_mlp_bn(edge_attr, params['edge1']), ep1)
    temp = _mlp_bn(jnp.concatenate([edge_attr3, edge_attr4], axis=1),
                   params['edge2'])
    e3a = _pad_to(temp, eph)
    e3b = e3a

    cnt1 = _sc_count((dst1_sc,), ones_chunk, zeros_acc)
    c1a, c1b = cnt1[0, :_N], cnt1[1, :_N]
    cnt3 = _sc_count((dst3_sc, src3_sc), ones_chunk, zeros_acc)
    c3a, c3b = cnt3[0, :_N], cnt3[1, :_N]

    for p in params['conv1']:
        wcat = jnp.concatenate([p['Wm'], p['We']], axis=1)
        bcat = jnp.concatenate([p['bm'], p['be']])[None, :]
        xs, xd = _sc_gather2(out, src1, dst1)
        msg, e = _msg_edge(xs, xd, e, wcat, bcat)
        parts = _sc_scatter_add(msg, dst1_sc, zeros_acc)
        out = _update(out, parts[0, :_N], parts[1, :_N], c1a, c1b,
                      p['Wu'], p['bu'][None, :])

    for p in params['conv2']:
        wcat = jnp.concatenate([p['Wm'], p['We']], axis=1)
        bcat = jnp.concatenate([p['bm'], p['be']])[None, :]
        gs, gd = _sc_gather2(out, src3, dst3)
        msg_f, msg_b, e3a, e3b = _msg_edge2(gs, gd, e3a, e3b, wcat, bcat)
        parts = _sc_scatter_add2(msg_f, msg_b, dst3_sc, src3_sc, zeros_acc)
        out = _update(out, parts[0, :_N], parts[1, :_N], c3a, c3b,
                      p['Wu'], p['bu'][None, :])

    xs, xd = _sc_gather2(out, src3, dst3)
    hp = params['head']
    yh = _head(xs, xd, e3a, _pad_to(edge_attr3, eph),
               hp['W1'], hp['b1'], hp['W2'], hp['b2'])
    return yh[:n_e3, 0]


# speedup vs baseline: 1.6732x; 1.5314x over previous
import functools

import jax
import jax.numpy as jnp
from jax import lax
from jax.experimental import pallas as pl
from jax.experimental.pallas import tpu as pltpu
from jax.experimental.pallas import tpu_sc as plsc

_DIM = 128
_N = 10000
_NPAD = 10240
_DUMMY = 10000
_NC, _NS = 2, 16
_NW = _NC * _NS
_CH = 128
_CHG = 64


def _pick_block(n):
    for b in (512, 400, 320, 256, 200, 160, 128, 80, 64, 40, 32, 16, 8):
        if n % b == 0:
            return b
    raise ValueError(f"no row block for {n}")


def _stats_body(x_ref, o_ref):
    xb = x_ref[...]
    s = jnp.sum(xb, axis=0, keepdims=True)
    q = jnp.sum(xb * xb, axis=0, keepdims=True)

    @pl.when(pl.program_id(0) == 0)
    def _():
        o_ref[...] = jnp.zeros_like(o_ref)

    o_ref[...] += jnp.concatenate([s, q], axis=0)


def _col_stats(x):
    n, d = x.shape
    b = _pick_block(n)
    out = pl.pallas_call(
        _stats_body,
        grid=(n // b,),
        in_specs=[pl.BlockSpec((b, d), lambda i: (i, 0))],
        out_specs=pl.BlockSpec((2, d), lambda i: (0, 0)),
        out_shape=jax.ShapeDtypeStruct((2, d), jnp.float32),
    )(x)
    mu = out[0] / n
    var = out[1] / n - mu * mu
    return mu, var


def _fold_bn(mu, var, g, be, w, b):
    a = g * jax.lax.rsqrt(var + 1e-5)
    wf = w * a[:, None]
    bf = (be - mu * a) @ w + b
    return wf, bf[None, :]


def _mm_body(x_ref, w_ref, b_ref, o_ref, *, alpha):
    h = jnp.dot(x_ref[...], w_ref[...], preferred_element_type=jnp.float32)
    h = h + b_ref[...]
    o_ref[...] = jnp.where(h >= 0, h, alpha * h)


def _mm_stats_body(x_ref, w_ref, b_ref, o_ref, st_ref, *, alpha):
    h = jnp.dot(x_ref[...], w_ref[...], preferred_element_type=jnp.float32)
    h = h + b_ref[...]
    y = jnp.where(h >= 0, h, alpha * h)
    o_ref[...] = y

    @pl.when(pl.program_id(0) == 0)
    def _():
        st_ref[...] = jnp.zeros_like(st_ref)

    s = jnp.sum(y, axis=0, keepdims=True)
    q = jnp.sum(y * y, axis=0, keepdims=True)
    st_ref[...] += jnp.concatenate([s, q], axis=0)


def _mm_act(x, w, b, alpha, want_stats=False):
    n, din = x.shape
    dout = w.shape[1]
    bl = _pick_block(n)
    grid = (n // bl,)
    in_specs = [
        pl.BlockSpec((bl, din), lambda i: (i, 0)),
        pl.BlockSpec((din, dout), lambda i: (0, 0)),
        pl.BlockSpec((1, dout), lambda i: (0, 0)),
    ]
    if want_stats:
        y, st = pl.pallas_call(
            functools.partial(_mm_stats_body, alpha=alpha),
            grid=grid,
            in_specs=in_specs,
            out_specs=(
                pl.BlockSpec((bl, dout), lambda i: (i, 0)),
                pl.BlockSpec((2, dout), lambda i: (0, 0)),
            ),
            out_shape=(
                jax.ShapeDtypeStruct((n, dout), jnp.float32),
                jax.ShapeDtypeStruct((2, dout), jnp.float32),
            ),
        )(x, w, b)
        mu = st[0] / n
        var = st[1] / n - mu * mu
        return y, mu, var
    return pl.pallas_call(
        functools.partial(_mm_body, alpha=alpha),
        grid=grid,
        in_specs=in_specs,
        out_specs=pl.BlockSpec((bl, dout), lambda i: (i, 0)),
        out_shape=jax.ShapeDtypeStruct((n, dout), jnp.float32),
    )(x, w, b)


def _mlp_bn(x, p):
    mu1, var1 = _col_stats(x)
    w1, b1 = _fold_bn(mu1, var1, p['g1'], p['be1'], p['W1'], p['b1'])
    y, mu2, var2 = _mm_act(x, w1, b1, 0.01, want_stats=True)
    w2, b2 = _fold_bn(mu2, var2, p['g2'], p['be2'], p['W2'], p['b2'])
    return _mm_act(y, w2, b2, 0.01)


def _msg_body(xs_ref, xd_ref, e_ref, w_ref, b_ref, msg_ref, en_ref):
    h = jnp.concatenate([xs_ref[...], xd_ref[...], e_ref[...]], axis=1)
    y = jnp.dot(h, w_ref[...], preferred_element_type=jnp.float32) + b_ref[...]
    y = jnp.maximum(y, 0.0)
    msg_ref[...] = y[:, :_DIM]
    en_ref[...] = y[:, _DIM:]


def _msg_edge(xs, xd, e, wcat, bcat):
    n = xs.shape[0]
    bl = _pick_block(n)
    return pl.pallas_call(
        _msg_body,
        grid=(n // bl,),
        in_specs=[
            pl.BlockSpec((bl, _DIM), lambda i: (i, 0)),
            pl.BlockSpec((bl, _DIM), lambda i: (i, 0)),
            pl.BlockSpec((bl, _DIM), lambda i: (i, 0)),
            pl.BlockSpec((3 * _DIM, 2 * _DIM), lambda i: (0, 0)),
            pl.BlockSpec((1, 2 * _DIM), lambda i: (0, 0)),
        ],
        out_specs=(
            pl.BlockSpec((bl, _DIM), lambda i: (i, 0)),
            pl.BlockSpec((bl, _DIM), lambda i: (i, 0)),
        ),
        out_shape=(
            jax.ShapeDtypeStruct((n, _DIM), jnp.float32),
            jax.ShapeDtypeStruct((n, _DIM), jnp.float32),
        ),
    )(xs, xd, e, wcat, bcat)


def _msg2_body(gs_ref, gd_ref, ea_ref, eb_ref, w_ref, b_ref,
               mf_ref, mb_ref, ef_ref, eb2_ref):
    xs = gs_ref[...]
    xd = gd_ref[...]
    w = w_ref[...]
    b = b_ref[...]
    hf = jnp.concatenate([xs, xd, ea_ref[...]], axis=1)
    yf = jnp.maximum(jnp.dot(hf, w, preferred_element_type=jnp.float32) + b, 0.0)
    mf_ref[...] = yf[:, :_DIM]
    ef_ref[...] = yf[:, _DIM:]
    hb = jnp.concatenate([xd, xs, eb_ref[...]], axis=1)
    yb = jnp.maximum(jnp.dot(hb, w, preferred_element_type=jnp.float32) + b, 0.0)
    mb_ref[...] = yb[:, :_DIM]
    eb2_ref[...] = yb[:, _DIM:]


def _msg_edge2(gs, gd, ea, eb, wcat, bcat):
    n = gs.shape[0]
    bl = _pick_block(n)
    rspec = pl.BlockSpec((bl, _DIM), lambda i: (i, 0))
    return pl.pallas_call(
        _msg2_body,
        grid=(n // bl,),
        in_specs=[
            rspec, rspec, rspec, rspec,
            pl.BlockSpec((3 * _DIM, 2 * _DIM), lambda i: (0, 0)),
            pl.BlockSpec((1, 2 * _DIM), lambda i: (0, 0)),
        ],
        out_specs=(rspec, rspec, rspec, rspec),
        out_shape=tuple(jax.ShapeDtypeStruct((n, _DIM), jnp.float32)
                        for _ in range(4)),
    )(gs, gd, ea, eb, wcat, bcat)


def _upd_body(x_ref, a0_ref, a1_ref, c0_ref, c1_ref, w_ref, b_ref, o_ref):
    cnt = jnp.maximum(c0_ref[:, 0:1] + c1_ref[:, 0:1], 1.0)
    agg = (a0_ref[...] + a1_ref[...]) / cnt
    h = jnp.dot(agg, w_ref[...], preferred_element_type=jnp.float32)
    h = x_ref[...] + h + b_ref[...]
    o_ref[...] = jnp.maximum(h, 0.0)


def _update(x, a0, a1, c0, c1, wu, bu):
    n = x.shape[0]
    bl = _pick_block(n)
    return pl.pallas_call(
        _upd_body,
        grid=(n // bl,),
        in_specs=[
            pl.BlockSpec((bl, _DIM), lambda i: (i, 0)),
            pl.BlockSpec((bl, _DIM), lambda i: (i, 0)),
            pl.BlockSpec((bl, _DIM), lambda i: (i, 0)),
            pl.BlockSpec((bl, _DIM), lambda i: (i, 0)),
            pl.BlockSpec((bl, _DIM), lambda i: (i, 0)),
            pl.BlockSpec((_DIM, _DIM), lambda i: (0, 0)),
            pl.BlockSpec((1, _DIM), lambda i: (0, 0)),
        ],
        out_specs=pl.BlockSpec((bl, _DIM), lambda i: (i, 0)),
        out_shape=jax.ShapeDtypeStruct((n, _DIM), jnp.float32),
    )(x, a0, a1, c0, c1, wu, bu)


def _head_body(xs_ref, xd_ref, e3_ref, ea_ref, ws_ref, wd_ref, we_ref,
               wa_ref, b1_ref, w2_ref, b2_ref, o_ref):
    h = (jnp.dot(xs_ref[...], ws_ref[...], preferred_element_type=jnp.float32)
         + jnp.dot(xd_ref[...], wd_ref[...], preferred_element_type=jnp.float32)
         + jnp.dot(e3_ref[...], we_ref[...], preferred_element_type=jnp.float32)
         + jnp.dot(ea_ref[...], wa_ref[...], preferred_element_type=jnp.float32)
         + b1_ref[...])
    h = jnp.maximum(h, 0.0)
    y = jnp.dot(h, w2_ref[...], preferred_element_type=jnp.float32)
    o_ref[...] = y + b2_ref[:, 0:1]


def _head(xs, xd, e3, ea, w1, b1, w2, b2):
    n = xs.shape[0]
    bl = _pick_block(n)
    d2 = 2 * _DIM
    ea_d = ea.shape[1]
    return pl.pallas_call(
        _head_body,
        grid=(n // bl,),
        in_specs=[
            pl.BlockSpec((bl, _DIM), lambda i: (i, 0)),
            pl.BlockSpec((bl, _DIM), lambda i: (i, 0)),
            pl.BlockSpec((bl, _DIM), lambda i: (i, 0)),
            pl.BlockSpec((bl, ea_d), lambda i: (i, 0)),
            pl.BlockSpec((_DIM, d2), lambda i: (0, 0)),
            pl.BlockSpec((_DIM, d2), lambda i: (0, 0)),
            pl.BlockSpec((_DIM, d2), lambda i: (0, 0)),
            pl.BlockSpec((ea_d, d2), lambda i: (0, 0)),
            pl.BlockSpec((1, d2), lambda i: (0, 0)),
            pl.BlockSpec((d2, 1), lambda i: (0, 0)),
            pl.BlockSpec((1, _DIM), lambda i: (0, 0)),
        ],
        out_specs=pl.BlockSpec((bl, 1), lambda i: (i, 0)),
        out_shape=jax.ShapeDtypeStruct((n, 1), jnp.float32),
    )(xs, xd, e3, ea, w1[:_DIM], w1[_DIM:2 * _DIM], w1[2 * _DIM:3 * _DIM],
      w1[3 * _DIM:], b1[None, :], w2, jnp.broadcast_to(b2[None, :], (1, _DIM)))


def _sc_gather2(table, idx_s, idx_d):
    epad = idx_s.shape[0]
    per_w = epad // _NW
    nch = per_w // _CHG
    njj = nch // 2
    mesh = plsc.VectorSubcoreMesh(core_axis_name="c", subcore_axis_name="s")

    @functools.partial(
        pl.kernel, mesh=mesh,
        out_type=(jax.ShapeDtypeStruct((epad, _DIM), jnp.float32),
                  jax.ShapeDtypeStruct((epad, _DIM), jnp.float32)),
        scratch_types=[
            pltpu.VMEM((_CHG,), jnp.int32),
            pltpu.VMEM((_CHG,), jnp.int32),
            pltpu.VMEM((_CHG,), jnp.int32),
            pltpu.VMEM((_CHG,), jnp.int32),
            pltpu.VMEM((_CHG, _DIM), jnp.float32),
            pltpu.VMEM((_CHG, _DIM), jnp.float32),
            pltpu.VMEM((_CHG, _DIM), jnp.float32),
            pltpu.VMEM((_CHG, _DIM), jnp.float32),
            pltpu.VMEM_SHARED((_N, _DIM), jnp.float32),
        ] + [pltpu.SemaphoreType.DMA] * 4,
    )
    def k(table_hbm, idxs_hbm, idxd_hbm, outs_hbm, outd_hbm,
          is0, is1, id0, id1, bs0, bs1, bd0, bd1, tbl,
          gs0, gs1, gd0, gd1):
        s = lax.axis_index("s")
        wid = s * _NC + lax.axis_index("c")
        base = pl.multiple_of(wid * per_w, 8)

        @pl.when(s == 0)
        def _():
            pltpu.sync_copy(table_hbm, tbl)

        def _irow(c):
            return pl.ds(pl.multiple_of(base + c * _CHG, 8), _CHG)

        pltpu.sync_copy(idxs_hbm.at[_irow(0)], is0)
        pltpu.sync_copy(idxd_hbm.at[_irow(0)], id0)
        plsc.subcore_barrier()
        pltpu.async_copy(tbl.at[is0], bs0, gs0)
        pltpu.async_copy(tbl.at[id0], bd0, gd0)

        def body(j, _):
            e = 2 * j
            o = e + 1
            pltpu.sync_copy(idxs_hbm.at[_irow(o)], is1)
            pltpu.sync_copy(idxd_hbm.at[_irow(o)], id1)
            pltpu.async_copy(tbl.at[is1], bs1, gs1)
            pltpu.async_copy(tbl.at[id1], bd1, gd1)
            pltpu.make_async_copy(tbl.at[is0], bs0, gs0).wait()
            pltpu.make_async_copy(tbl.at[id0], bd0, gd0).wait()
            pltpu.sync_copy(bs0, outs_hbm.at[_irow(e)])
            pltpu.sync_copy(bd0, outd_hbm.at[_irow(e)])

            @pl.when(j + 1 < njj)
            def _():
                pltpu.sync_copy(idxs_hbm.at[_irow(e + 2)], is0)
                pltpu.sync_copy(idxd_hbm.at[_irow(e + 2)], id0)
                pltpu.async_copy(tbl.at[is0], bs0, gs0)
                pltpu.async_copy(tbl.at[id0], bd0, gd0)

            pltpu.make_async_copy(tbl.at[is1], bs1, gs1).wait()
            pltpu.make_async_copy(tbl.at[id1], bd1, gd1).wait()
            pltpu.sync_copy(bs1, outs_hbm.at[_irow(o)])
            pltpu.sync_copy(bd1, outd_hbm.at[_irow(o)])
            return 0

        lax.fori_loop(0, njj, body, 0, unroll=False)

    return k(table, idx_s, idx_d)


def _sc_scatter_add(msg, idx, zeros_acc):
    epad = idx.shape[0]
    mesh = plsc.VectorSubcoreMesh(core_axis_name="c", subcore_axis_name="s")
    rows_per_tile = _NPAD // _NS

    per_w = epad // _NW
    nch = per_w // _CH
    njj = nch // 2

    @functools.partial(
        pl.kernel, mesh=mesh,
        out_type=jax.ShapeDtypeStruct((_NC, _NPAD, _DIM), jnp.float32),
        scratch_types=[
            pltpu.VMEM((2, _CH), jnp.int32),
            pltpu.VMEM((_CH, _DIM), jnp.float32),
            pltpu.VMEM((_CH, _DIM), jnp.float32),
            pltpu.VMEM_SHARED((_NPAD, _DIM), jnp.float32),
            pltpu.SemaphoreType.DMA,
            pltpu.SemaphoreType.DMA,
            pltpu.SemaphoreType.DMA,
            pltpu.SemaphoreType.DMA,
        ],
    )
    def k(msg_hbm, idx_hbm, z_hbm, out_hbm, i2, m0, m1, acc, l0, l1, i0, i1):
        c = lax.axis_index("c")
        s = lax.axis_index("s")
        wid = s * _NC + c
        base = pl.multiple_of(wid * per_w, 8)

        @pl.when(s == 0)
        def _():
            pltpu.sync_copy(z_hbm, acc)

        def _row(ch):
            return pl.ds(pl.multiple_of(base + ch * _CH, 8), _CH)

        plsc.subcore_barrier()
        pltpu.async_copy(msg_hbm.at[_row(0)], m0, l0)
        pltpu.async_copy(idx_hbm.at[_row(0)], i2.at[0], i0)

        def body(j, _):
            e = 2 * j
            o = e + 1
            pltpu.make_async_copy(msg_hbm.at[_row(e)], m0, l0).wait()
            pltpu.make_async_copy(idx_hbm.at[_row(e)], i2.at[0], i0).wait()
            pltpu.async_copy(msg_hbm.at[_row(o)], m1, l1)
            pltpu.async_copy(idx_hbm.at[_row(o)], i2.at[1], i1)
            pltpu.sync_copy(m0, acc.at[i2.at[0]], add=True)
            pltpu.make_async_copy(msg_hbm.at[_row(o)], m1, l1).wait()
            pltpu.make_async_copy(idx_hbm.at[_row(o)], i2.at[1], i1).wait()

            @pl.when(j + 1 < njj)
            def _():
                pltpu.async_copy(msg_hbm.at[_row(e + 2)], m0, l0)
                pltpu.async_copy(idx_hbm.at[_row(e + 2)], i2.at[0], i0)

            pltpu.sync_copy(m1, acc.at[i2.at[1]], add=True)
            return 0

        lax.fori_loop(0, njj, body, 0, unroll=False)
        plsc.subcore_barrier()
        orow = pl.ds(pl.multiple_of(s * rows_per_tile, 8), rows_per_tile)
        pltpu.sync_copy(acc.at[orow], out_hbm.at[c, orow])

    return k(msg, idx, zeros_acc)


def _sc_scatter_add2(msg_a, msg_b, idx_a, idx_b, zeros_acc):
    epad = idx_a.shape[0]
    mesh = plsc.VectorSubcoreMesh(core_axis_name="c", subcore_axis_name="s")
    rows_per_tile = _NPAD // _NS
    per_w = epad // _NW
    nch = per_w // _CH
    njj = nch // 2

    @functools.partial(
        pl.kernel, mesh=mesh,
        out_type=jax.ShapeDtypeStruct((_NC, _NPAD, _DIM), jnp.float32),
        scratch_types=[
            pltpu.VMEM((2, _CH), jnp.int32),
            pltpu.VMEM((_CH, _DIM), jnp.float32),
            pltpu.VMEM((_CH, _DIM), jnp.float32),
            pltpu.VMEM_SHARED((_NPAD, _DIM), jnp.float32),
            pltpu.SemaphoreType.DMA,
            pltpu.SemaphoreType.DMA,
            pltpu.SemaphoreType.DMA,
            pltpu.SemaphoreType.DMA,
        ],
    )
    def k(msga_hbm, msgb_hbm, idxa_hbm, idxb_hbm, z_hbm, out_hbm,
          i2, m0, m1, acc, l0, l1, i0, i1):
        c = lax.axis_index("c")
        s = lax.axis_index("s")
        wid = s * _NC + c
        base = pl.multiple_of(wid * per_w, 8)

        @pl.when(s == 0)
        def _():
            pltpu.sync_copy(z_hbm, acc)

        def _row(ch):
            return pl.ds(pl.multiple_of(base + ch * _CH, 8), _CH)

        plsc.subcore_barrier()

        def _pass(msg_hbm, idx_hbm):
            pltpu.async_copy(msg_hbm.at[_row(0)], m0, l0)
            pltpu.async_copy(idx_hbm.at[_row(0)], i2.at[0], i0)

            def body(j, _):
                e = 2 * j
                o = e + 1
                pltpu.make_async_copy(msg_hbm.at[_row(e)], m0, l0).wait()
                pltpu.make_async_copy(idx_hbm.at[_row(e)], i2.at[0], i0).wait()
                pltpu.async_copy(msg_hbm.at[_row(o)], m1, l1)
                pltpu.async_copy(idx_hbm.at[_row(o)], i2.at[1], i1)
                pltpu.sync_copy(m0, acc.at[i2.at[0]], add=True)
                pltpu.make_async_copy(msg_hbm.at[_row(o)], m1, l1).wait()
                pltpu.make_async_copy(idx_hbm.at[_row(o)], i2.at[1], i1).wait()

                @pl.when(j + 1 < njj)
                def _():
                    pltpu.async_copy(msg_hbm.at[_row(e + 2)], m0, l0)
                    pltpu.async_copy(idx_hbm.at[_row(e + 2)], i2.at[0], i0)

                pltpu.sync_copy(m1, acc.at[i2.at[1]], add=True)
                return 0

            lax.fori_loop(0, njj, body, 0, unroll=False)

        _pass(msga_hbm, idxa_hbm)
        _pass(msgb_hbm, idxb_hbm)
        plsc.subcore_barrier()
        orow = pl.ds(pl.multiple_of(s * rows_per_tile, 8), rows_per_tile)
        pltpu.sync_copy(acc.at[orow], out_hbm.at[c, orow])

    return k(msg_a, msg_b, idx_a, idx_b, zeros_acc)


def _sc_count(idxs, ones_chunk, zeros_cnt):
    idxs = tuple(idxs)
    epad = idxs[0].shape[0]
    per_w = epad // _NW
    nch = per_w // _CH
    njj = nch // 2
    mesh = plsc.VectorSubcoreMesh(core_axis_name="c", subcore_axis_name="s")
    rows_per_tile = _NPAD // _NS

    @functools.partial(
        pl.kernel, mesh=mesh,
        out_type=jax.ShapeDtypeStruct((_NC, _NPAD, _DIM), jnp.float32),
        scratch_types=[
            pltpu.VMEM((2, _CH), jnp.int32),
            pltpu.VMEM((_CH, _DIM), jnp.float32),
            pltpu.VMEM_SHARED((_NPAD, _DIM), jnp.float32),
            pltpu.SemaphoreType.DMA,
            pltpu.SemaphoreType.DMA,
        ],
    )
    def k(*refs):
        idx_hbms = refs[:len(idxs)]
        ones_hbm, z_hbm, out_hbm, i2, ones_v, acc, i0, i1 = refs[len(idxs):]
        c = lax.axis_index("c")
        s = lax.axis_index("s")
        wid = s * _NC + c
        base = pl.multiple_of(wid * per_w, 8)

        @pl.when(s == 0)
        def _():
            pltpu.sync_copy(z_hbm, acc)

        def _row(ch):
            return pl.ds(pl.multiple_of(base + ch * _CH, 8), _CH)

        pltpu.sync_copy(ones_hbm, ones_v)
        plsc.subcore_barrier()

        def _pass(idx_hbm):
            pltpu.async_copy(idx_hbm.at[_row(0)], i2.at[0], i0)

            def body(j, _):
                e = 2 * j
                o = e + 1
                pltpu.make_async_copy(idx_hbm.at[_row(e)], i2.at[0], i0).wait()
                pltpu.async_copy(idx_hbm.at[_row(o)], i2.at[1], i1)
                pltpu.sync_copy(ones_v, acc.at[i2.at[0]], add=True)
                pltpu.make_async_copy(idx_hbm.at[_row(o)], i2.at[1], i1).wait()

                @pl.when(j + 1 < njj)
                def _():
                    pltpu.async_copy(idx_hbm.at[_row(e + 2)], i2.at[0], i0)

                pltpu.sync_copy(ones_v, acc.at[i2.at[1]], add=True)
                return 0

            lax.fori_loop(0, njj, body, 0, unroll=False)

        for idx_hbm in idx_hbms:
            _pass(idx_hbm)
        plsc.subcore_barrier()
        orow = pl.ds(pl.multiple_of(s * rows_per_tile, 8), rows_per_tile)
        pltpu.sync_copy(acc.at[orow], out_hbm.at[c, orow])

    return k(*idxs, ones_chunk, zeros_cnt)


def _pad_to(a, n, fill=0):
    pad = n - a.shape[0]
    if pad == 0:
        return a
    shape = (pad,) + a.shape[1:]
    return jnp.concatenate([a, jnp.full(shape, fill, a.dtype)], axis=0)


def _round_up(n, m):
    return ((n + m - 1) // m) * m


def kernel(x, edge_attr, edge_attr3, edge_attr4, params, edge_index, edge_index3):
    n_e = edge_index.shape[1]
    n_e3 = edge_index3.shape[1]
    ep1 = _round_up(n_e, 2 * _NW * _CH)
    eph = _round_up(n_e3, 2 * _NW * _CH)

    zeros_acc = jnp.zeros((_NPAD, _DIM), jnp.float32)
    ones_chunk = jnp.ones((_CH, _DIM), jnp.float32)

    src1 = _pad_to(edge_index[0], ep1)
    dst1 = _pad_to(edge_index[1], ep1)
    dst1_sc = _pad_to(edge_index[1], ep1, _DUMMY)

    src3 = _pad_to(edge_index3[0], eph)
    dst3 = _pad_to(edge_index3[1], eph)
    src3_sc = _pad_to(edge_index3[0], eph, _DUMMY)
    dst3_sc = _pad_to(edge_index3[1], eph, _DUMMY)

    out = _mlp_bn(x, params['node'])
    e = _pad_to(_mlp_bn(edge_attr, params['edge1']), ep1)
    temp = _mlp_bn(jnp.concatenate([edge_attr3, edge_attr4], axis=1),
                   params['edge2'])
    e3a = _pad_to(temp, eph)
    e3b = e3a

    cnt1 = _sc_count((dst1_sc,), ones_chunk, zeros_acc)
    c1a, c1b = cnt1[0, :_N], cnt1[1, :_N]
    cnt3 = _sc_count((dst3_sc, src3_sc), ones_chunk, zeros_acc)
    c3a, c3b = cnt3[0, :_N], cnt3[1, :_N]

    for p in params['conv1']:
        wcat = jnp.concatenate([p['Wm'], p['We']], axis=1)
        bcat = jnp.concatenate([p['bm'], p['be']])[None, :]
        xs, xd = _sc_gather2(out, src1, dst1)
        msg, e = _msg_edge(xs, xd, e, wcat, bcat)
        parts = _sc_scatter_add(msg, dst1_sc, zeros_acc)
        out = _update(out, parts[0, :_N], parts[1, :_N], c1a, c1b,
                      p['Wu'], p['bu'][None, :])

    for p in params['conv2']:
        wcat = jnp.concatenate([p['Wm'], p['We']], axis=1)
        bcat = jnp.concatenate([p['bm'], p['be']])[None, :]
        gs, gd = _sc_gather2(out, src3, dst3)
        msg_f, msg_b, e3a, e3b = _msg_edge2(gs, gd, e3a, e3b, wcat, bcat)
        parts = _sc_scatter_add2(msg_f, msg_b, dst3_sc, src3_sc, zeros_acc)
        out = _update(out, parts[0, :_N], parts[1, :_N], c3a, c3b,
                      p['Wu'], p['bu'][None, :])

    xs, xd = _sc_gather2(out, src3, dst3)
    hp = params['head']
    yh = _head(xs, xd, e3a, _pad_to(edge_attr3, eph),
               hp['W1'], hp['b1'], hp['W2'], hp['b2'])
    return yh[:n_e3, 0]
